# Initial kernel scaffold; baseline (speedup 1.0000x reference)
import jax
import jax.numpy as jnp
from jax.experimental import pallas as pl


def kernel(x, node_type, embed):
    raise NotImplementedError("write your pallas kernel here")


# speedup vs baseline: 5.4548x; 5.4548x over previous
import functools

import jax
import jax.numpy as jnp
from jax.experimental import pallas as pl

N = 2048
DIM = 128
NC = 2
K = 512
HD = DIM // NC
BN = 512


def _vq_kernel(x_ref, embed_ref, q_ref, idx_ref):
    x = x_ref[...]
    lane_iota = jax.lax.broadcasted_iota(jnp.int32, (BN, K), 1)
    idx_cols = []
    q_cols = []
    for c in range(NC):
        xc = x[:, c * HD:(c + 1) * HD]
        ec = embed_ref[c]
        ecT = jnp.transpose(ec)
        s = jax.lax.dot_general(
            xc, ecT, (((1,), (0,)), ((), ())),
            preferred_element_type=jnp.float32,
            precision=jax.lax.Precision.HIGHEST)
        en = jnp.sum(ecT * ecT, axis=0, keepdims=True)
        d = en - 2.0 * s
        m1 = jnp.min(d, axis=1, keepdims=True)
        i1 = jnp.min(jnp.where(d == m1, lane_iota, K), axis=1,
                     keepdims=True)
        d2m = jnp.where(lane_iota == i1, jnp.inf, d)
        m2 = jnp.min(d2m, axis=1, keepdims=True)
        i2 = jnp.min(jnp.where(d2m == m2, lane_iota, K), axis=1,
                     keepdims=True)
        oh1 = (lane_iota == i1).astype(jnp.float32)
        oh2 = (lane_iota == i2).astype(jnp.float32)
        e1 = jax.lax.dot_general(
            oh1, ec, (((1,), (0,)), ((), ())),
            preferred_element_type=jnp.float32,
            precision=jax.lax.Precision.HIGHEST)
        e2 = jax.lax.dot_general(
            oh2, ec, (((1,), (0,)), ((), ())),
            preferred_element_type=jnp.float32,
            precision=jax.lax.Precision.HIGHEST)
        r1 = xc - e1
        r2 = xc - e2
        d1 = jnp.sum(r1 * r1, axis=1, keepdims=True)
        d2 = jnp.sum(r2 * r2, axis=1, keepdims=True)
        take2 = (d2 < d1) | ((d2 == d1) & (i2 < i1))
        idx_cols.append(jnp.where(take2, i2, i1))
        q_cols.append(jnp.where(take2, e2, e1))
    q_ref[...] = jnp.concatenate(q_cols, axis=1)
    idx_ref[...] = jnp.concatenate(idx_cols, axis=1)


@jax.jit
def kernel(x, node_type, embed):
    del node_type
    grid = (N // BN,)
    q, idx = pl.pallas_call(
        _vq_kernel,
        grid=grid,
        in_specs=[
            pl.BlockSpec((BN, DIM), lambda i: (i, 0)),
            pl.BlockSpec((NC, K, HD), lambda i: (0, 0, 0)),
        ],
        out_specs=[
            pl.BlockSpec((BN, DIM), lambda i: (i, 0)),
            pl.BlockSpec((BN, NC), lambda i: (i, 0)),
        ],
        out_shape=[
            jax.ShapeDtypeStruct((N, DIM), jnp.float32),
            jax.ShapeDtypeStruct((N, NC), jnp.int32),
        ],
    )(x, embed)
    return (q, idx, 0)
